# baseline (speedup 1.0000x reference)
import functools

import jax
import jax.numpy as jnp
from jax import lax
from jax.experimental import pallas as pl
from jax.experimental.pallas import tpu as pltpu
from jax.experimental.pallas import tpu_sc as plsc

_NC = 2
_NS = 16
_NW = _NC * _NS

_B = 16384
_D = 128
_CHUNK = 128
_ROWS_PER_W = _B // _NW
_CPW = _ROWS_PER_W // _CHUNK


def _gather_body(uidx_hbm, iidx_hbm, utab_hbm, itab_hbm, ue_out, ie_out,
                 idx_u, idx_i, rows, sem):
    wid = lax.axis_index("s") * _NC + lax.axis_index("c")
    row0 = wid * _ROWS_PER_W
    for j in range(_CPW):
        pltpu.sync_copy(uidx_hbm.at[pl.ds(row0 + j * _CHUNK, _CHUNK)],
                        idx_u.at[j])
        pltpu.sync_copy(iidx_hbm.at[pl.ds(row0 + j * _CHUNK, _CHUNK)],
                        idx_i.at[j])
    cps = [pltpu.async_copy(utab_hbm.at[idx_u.at[j]], rows.at[j], sem)
           for j in range(_CPW)]
    for c in cps:
        c.wait()
    for j in range(_CPW):
        pltpu.sync_copy(rows.at[j], ue_out.at[pl.ds(row0 + j * _CHUNK, _CHUNK)])
    cps = [pltpu.async_copy(itab_hbm.at[idx_i.at[j]], rows.at[j], sem)
           for j in range(_CPW)]
    for c in cps:
        c.wait()
    for j in range(_CPW):
        pltpu.sync_copy(rows.at[j], ie_out.at[pl.ds(row0 + j * _CHUNK, _CHUNK)])


@jax.jit
def _sc_gather(uidx, iidx, user_table, item_table):
    mesh = plsc.VectorSubcoreMesh(core_axis_name="c", subcore_axis_name="s",
                                  num_cores=_NC, num_subcores=_NS)
    grab = pl.kernel(
        _gather_body,
        out_type=[
            jax.ShapeDtypeStruct((_B, _D), jnp.float32),
            jax.ShapeDtypeStruct((_B, _D), jnp.float32),
        ],
        mesh=mesh,
        scratch_types=[
            pltpu.VMEM((_CPW, _CHUNK), jnp.int32),
            pltpu.VMEM((_CPW, _CHUNK), jnp.int32),
            pltpu.VMEM((_CPW, _CHUNK, _D), jnp.float32),
            pltpu.SemaphoreType.DMA,
        ],
        compiler_params=pltpu.CompilerParams(use_tc_tiling_on_sc=True),
        name="ncf_sc_gather",
    )
    return grab(uidx, iidx, user_table, item_table)


def _mlp_body(ue_ref, ie_ref, w1a, w1b, b1, w2, b2, w3, b3, wo, bo, out_ref):
    h = jnp.maximum(
        ue_ref[...] @ w1a[...] + ie_ref[...] @ w1b[...] + b1[...], 0.0)
    h = jnp.maximum(h @ w2[...] + b2[...], 0.0)
    h = jnp.maximum(h @ w3[...] + b3[...], 0.0)
    out_ref[...] = h @ wo[...] + bo[...]


_BM = 1024


@jax.jit
def _tc_mlp(ue, ie, w1a, w1b, b1, w2, b2, w3, b3, wo, bo):
    full = lambda shape: pl.BlockSpec(shape, lambda i: (0, 0))
    return pl.pallas_call(
        _mlp_body,
        grid=(_B // _BM,),
        in_specs=[
            pl.BlockSpec((_BM, _D), lambda i: (i, 0)),
            pl.BlockSpec((_BM, _D), lambda i: (i, 0)),
            full((128, 128)), full((128, 128)), full((1, 128)),
            full((128, 64)), full((1, 64)),
            full((64, 32)), full((1, 32)),
            full((32, 1)), full((1, 1)),
        ],
        out_specs=pl.BlockSpec((_BM, 1), lambda i: (i, 0)),
        out_shape=jax.ShapeDtypeStruct((_B, 1), jnp.float32),
        name="ncf_tc_mlp",
    )(ue, ie, w1a, w1b, b1, w2, b2, w3, b3, wo, bo)


def kernel(user_indices, item_indices, user_table, item_table,
           W1, b1, W2, b2, W3, b3, Wo, bo):
    ue, ie = _sc_gather(user_indices, item_indices, user_table, item_table)
    return _tc_mlp(ue, ie,
                   W1[:_D], W1[_D:], b1.reshape(1, -1),
                   W2, b2.reshape(1, -1),
                   W3, b3.reshape(1, -1),
                   Wo, bo.reshape(1, -1))


# speedup vs baseline: 1.3239x; 1.3239x over previous
import functools

import jax
import jax.numpy as jnp
from jax import lax
from jax.experimental import pallas as pl
from jax.experimental.pallas import tpu as pltpu
from jax.experimental.pallas import tpu_sc as plsc

_NC = 2
_NS = 16
_NW = _NC * _NS

_B = 16384
_D = 128
_CHUNK = 128
_HALF = _B // 2


def _gather_body(cpw, uidx_hbm, iidx_hbm, utab_hbm, itab_hbm, ue_out, ie_out,
                 idx_u, idx_i, buf_a, buf_b, gsem, ssem):
    wid = lax.axis_index("s") * _NC + lax.axis_index("c")
    base = wid * cpw
    pltpu.sync_copy(uidx_hbm.at[pl.ds(base, cpw)], idx_u)
    pltpu.sync_copy(iidx_hbm.at[pl.ds(base, cpw)], idx_i)
    phases = []
    for idx, out in ((idx_u, ue_out), (idx_i, ie_out)):
        for s in range(cpw // 2):
            phases.append((idx, 2 * s, out))
    bufs = (buf_a, buf_b)
    tabs = {id(idx_u): utab_hbm, id(idx_i): itab_hbm}
    stores = [None, None]
    for p, (idx, j0, out) in enumerate(phases):
        buf = bufs[p % 2]
        if stores[p % 2] is not None:
            stores[p % 2].wait()
        tab = tabs[id(idx)]
        gs = [pltpu.async_copy(tab.at[idx.at[j0 + jj]], buf.at[jj], gsem)
              for jj in range(2)]
        for g in gs:
            g.wait()
        stores[p % 2] = pltpu.async_copy(
            buf, out.at[pl.ds(base + j0, 2)], ssem)
    for s in stores:
        if s is not None:
            s.wait()


def _sc_gather(uidx2d, iidx2d, user_table, item_table, nrows):
    cpw = nrows // _NW // _CHUNK
    mesh = plsc.VectorSubcoreMesh(core_axis_name="c", subcore_axis_name="s",
                                  num_cores=_NC, num_subcores=_NS)
    grab = pl.kernel(
        functools.partial(_gather_body, cpw),
        out_type=[
            jax.ShapeDtypeStruct((nrows // _CHUNK, _CHUNK, _D), jnp.float32),
            jax.ShapeDtypeStruct((nrows // _CHUNK, _CHUNK, _D), jnp.float32),
        ],
        mesh=mesh,
        scratch_types=[
            pltpu.VMEM((cpw, _CHUNK), jnp.int32),
            pltpu.VMEM((cpw, _CHUNK), jnp.int32),
            pltpu.VMEM((2, _CHUNK, _D), jnp.float32),
            pltpu.VMEM((2, _CHUNK, _D), jnp.float32),
            pltpu.SemaphoreType.DMA,
            pltpu.SemaphoreType.DMA,
        ],
        name="ncf_sc_gather",
    )
    return grab(uidx2d, iidx2d, user_table, item_table)


def _dgT(w, xT):
    return lax.dot_general(w, xT, (((0,), (0,)), ((), ())),
                           preferred_element_type=jnp.float32)


def _mlp_body(ue_ref, ie_ref, w1a, w1b, b1, w2, b2, w3, b3, wo, bo, out_ref):
    ueT = ue_ref[...].T
    ieT = ie_ref[...].T
    h = jnp.maximum(_dgT(w1a[...], ueT) + _dgT(w1b[...], ieT) + b1[...], 0.0)
    h = jnp.maximum(_dgT(w2[...], h) + b2[...], 0.0)
    h = jnp.maximum(_dgT(w3[...], h) + b3[...], 0.0)
    out_ref[...] = _dgT(wo[...], h) + bo[...]


_BM = 4096


def _tc_mlp(ue, ie, w1a, w1b, b1, w2, b2, w3, b3, wo, bo, nrows):
    full = lambda shape: pl.BlockSpec(shape, lambda i: (0, 0))
    return pl.pallas_call(
        _mlp_body,
        grid=(nrows // _BM,),
        in_specs=[
            pl.BlockSpec((_BM, _D), lambda i: (i, 0)),
            pl.BlockSpec((_BM, _D), lambda i: (i, 0)),
            full((128, 128)), full((128, 128)), full((128, 1)),
            full((128, 64)), full((64, 1)),
            full((64, 32)), full((32, 1)),
            full((32, 1)), full((1, 1)),
        ],
        out_specs=pl.BlockSpec((1, _BM), lambda i: (0, i)),
        out_shape=jax.ShapeDtypeStruct((1, nrows), jnp.float32),
        compiler_params=pltpu.CompilerParams(
            dimension_semantics=("parallel",)),
        name="ncf_tc_mlp",
    )(ue, ie, w1a, w1b, b1, w2, b2, w3, b3, wo, bo)


def kernel(user_indices, item_indices, user_table, item_table,
           W1, b1, W2, b2, W3, b3, Wo, bo):
    nch = _HALF // _CHUNK
    uidx2d = user_indices.reshape(_B // _CHUNK, _CHUNK)
    iidx2d = item_indices.reshape(_B // _CHUNK, _CHUNK)
    weights = (W1[:_D], W1[_D:], b1.reshape(-1, 1),
               W2, b2.reshape(-1, 1),
               W3, b3.reshape(-1, 1),
               Wo, bo.reshape(1, 1))
    outs = []
    halves = []
    for h in range(2):
        sl = slice(h * nch, (h + 1) * nch)
        halves.append(_sc_gather(uidx2d[sl], iidx2d[sl],
                                 user_table, item_table, _HALF))
    for ue3d, ie3d in halves:
        ue = ue3d.reshape(_HALF, _D)
        ie = ie3d.reshape(_HALF, _D)
        outs.append(_tc_mlp(ue, ie, *weights, _HALF))
    out = jnp.concatenate(outs, axis=1)
    return out.reshape(_B, 1)


# speedup vs baseline: 1.4595x; 1.1024x over previous
import functools

import jax
import jax.numpy as jnp
from jax import lax
from jax.experimental import pallas as pl
from jax.experimental.pallas import tpu as pltpu
from jax.experimental.pallas import tpu_sc as plsc

_NC = 2
_NS = 16
_NW = _NC * _NS

_B = 16384
_D = 128
_CHUNK = 128
_HALF = _B // 2


def _gather_body(cpw, uidx_hbm, iidx_hbm, utab_hbm, itab_hbm, ue_out, ie_out,
                 idx_u, idx_i, buf_a, buf_b, gsem, ssem):
    wid = lax.axis_index("s") * _NC + lax.axis_index("c")
    base = wid * cpw
    pltpu.sync_copy(uidx_hbm.at[pl.ds(base, cpw)], idx_u)
    pltpu.sync_copy(iidx_hbm.at[pl.ds(base, cpw)], idx_i)
    phases = []
    for idx, out in ((idx_u, ue_out), (idx_i, ie_out)):
        for s in range(cpw // 2):
            phases.append((idx, 2 * s, out))
    bufs = (buf_a, buf_b)
    tabs = {id(idx_u): utab_hbm, id(idx_i): itab_hbm}
    stores = [None, None]
    for p, (idx, j0, out) in enumerate(phases):
        buf = bufs[p % 2]
        if stores[p % 2] is not None:
            stores[p % 2].wait()
        tab = tabs[id(idx)]
        gs = [pltpu.async_copy(tab.at[idx.at[j0 + jj]], buf.at[jj], gsem)
              for jj in range(2)]
        for g in gs:
            g.wait()
        stores[p % 2] = pltpu.async_copy(
            buf, out.at[pl.ds(base + j0, 2)], ssem)
    for s in stores:
        if s is not None:
            s.wait()


def _sc_gather(uidx2d, iidx2d, user_table, item_table, nrows):
    cpw = nrows // _NW // _CHUNK
    mesh = plsc.VectorSubcoreMesh(core_axis_name="c", subcore_axis_name="s",
                                  num_cores=_NC, num_subcores=_NS)
    grab = pl.kernel(
        functools.partial(_gather_body, cpw),
        out_type=[
            jax.ShapeDtypeStruct((nrows // _CHUNK, _CHUNK, _D), jnp.float32),
            jax.ShapeDtypeStruct((nrows // _CHUNK, _CHUNK, _D), jnp.float32),
        ],
        mesh=mesh,
        scratch_types=[
            pltpu.VMEM((cpw, _CHUNK), jnp.int32),
            pltpu.VMEM((cpw, _CHUNK), jnp.int32),
            pltpu.VMEM((2, _CHUNK, _D), jnp.float32),
            pltpu.VMEM((2, _CHUNK, _D), jnp.float32),
            pltpu.SemaphoreType.DMA,
            pltpu.SemaphoreType.DMA,
        ],
        name="ncf_sc_gather",
    )
    return grab(uidx2d, iidx2d, user_table, item_table)


def _dgT(w, xT):
    return lax.dot_general(w, xT, (((0,), (0,)), ((), ())),
                           preferred_element_type=jnp.float32)


def _mlp_body(ue_ref, ie_ref, w1a, w1b, b1, w2, b2, w3, b3, wo, bo, out_ref):
    ueT = ue_ref[...].T
    ieT = ie_ref[...].T
    h = jnp.maximum(_dgT(w1a[...], ueT) + _dgT(w1b[...], ieT) + b1[...], 0.0)
    h = jnp.maximum(_dgT(w2[...], h) + b2[...], 0.0)
    h = jnp.maximum(_dgT(w3[...], h) + b3[...], 0.0)
    out_ref[...] = _dgT(wo[...], h) + bo[...]


_BM = 4096


def _tc_mlp(ue, ie, w1a, w1b, b1, w2, b2, w3, b3, wo, bo, nrows):
    full = lambda shape: pl.BlockSpec(shape, lambda i: (0, 0))
    return pl.pallas_call(
        _mlp_body,
        grid=(nrows // _BM,),
        in_specs=[
            pl.BlockSpec((_BM, _D), lambda i: (i, 0)),
            pl.BlockSpec((_BM, _D), lambda i: (i, 0)),
            full((128, 128)), full((128, 128)), full((128, 1)),
            full((128, 64)), full((64, 1)),
            full((64, 32)), full((32, 1)),
            full((32, 1)), full((1, 1)),
        ],
        out_specs=pl.BlockSpec((1, _BM), lambda i: (0, i)),
        out_shape=jax.ShapeDtypeStruct((1, nrows), jnp.float32),
        compiler_params=pltpu.CompilerParams(
            dimension_semantics=("parallel",)),
        name="ncf_tc_mlp",
    )(ue, ie, w1a, w1b, b1, w2, b2, w3, b3, wo, bo)


def kernel(user_indices, item_indices, user_table, item_table,
           W1, b1, W2, b2, W3, b3, Wo, bo):
    uidx2d = user_indices.reshape(_B // _CHUNK, _CHUNK)
    iidx2d = item_indices.reshape(_B // _CHUNK, _CHUNK)
    weights = (W1[:_D], W1[_D:], b1.reshape(-1, 1),
               W2, b2.reshape(-1, 1),
               W3, b3.reshape(-1, 1),
               Wo, bo.reshape(1, 1))
    ue3d, ie3d = _sc_gather(uidx2d, iidx2d, user_table, item_table, _B)
    ue = ue3d.reshape(_B, _D)
    ie = ie3d.reshape(_B, _D)
    out = _tc_mlp(ue, ie, *weights, _B)
    return out.reshape(_B, 1)


# speedup vs baseline: 1.4785x; 1.0130x over previous
import functools

import jax
import jax.numpy as jnp
from jax import lax
from jax.experimental import pallas as pl
from jax.experimental.pallas import tpu as pltpu
from jax.experimental.pallas import tpu_sc as plsc

_NC = 2
_NS = 16
_NW = _NC * _NS

_B = 16384
_D = 128
_CHUNK = 128
_HALF = _B // 2


def _gather_body(cpw, uidx_hbm, iidx_hbm, utab_hbm, itab_hbm, ue_out, ie_out,
                 idx_u, idx_i, buf_a, buf_b, gsem, ssem):
    wid = lax.axis_index("s") * _NC + lax.axis_index("c")
    base = wid * cpw
    pltpu.sync_copy(uidx_hbm.at[pl.ds(base, cpw)], idx_u)
    pltpu.sync_copy(iidx_hbm.at[pl.ds(base, cpw)], idx_i)
    phases = []
    for idx, out in ((idx_u, ue_out), (idx_i, ie_out)):
        for s in range(cpw // 2):
            phases.append((idx, 2 * s, out))
    bufs = (buf_a, buf_b)
    tabs = {id(idx_u): utab_hbm, id(idx_i): itab_hbm}
    stores = [None, None]
    for p, (idx, j0, out) in enumerate(phases):
        buf = bufs[p % 2]
        if stores[p % 2] is not None:
            stores[p % 2].wait()
        tab = tabs[id(idx)]
        gs = [pltpu.async_copy(tab.at[idx.at[j0 + jj]], buf.at[jj], gsem)
              for jj in range(2)]
        for g in gs:
            g.wait()
        stores[p % 2] = pltpu.async_copy(
            buf, out.at[pl.ds(base + j0, 2)], ssem)
    for s in stores:
        if s is not None:
            s.wait()


def _sc_gather(uidx2d, iidx2d, user_table, item_table, nrows):
    cpw = nrows // _NW // _CHUNK
    mesh = plsc.VectorSubcoreMesh(core_axis_name="c", subcore_axis_name="s",
                                  num_cores=_NC, num_subcores=_NS)
    grab = pl.kernel(
        functools.partial(_gather_body, cpw),
        out_type=[
            jax.ShapeDtypeStruct((nrows // _CHUNK, _CHUNK, _D), jnp.float32),
            jax.ShapeDtypeStruct((nrows // _CHUNK, _CHUNK, _D), jnp.float32),
        ],
        mesh=mesh,
        scratch_types=[
            pltpu.VMEM((cpw, _CHUNK), jnp.int32),
            pltpu.VMEM((cpw, _CHUNK), jnp.int32),
            pltpu.VMEM((2, _CHUNK, _D), jnp.float32),
            pltpu.VMEM((2, _CHUNK, _D), jnp.float32),
            pltpu.SemaphoreType.DMA,
            pltpu.SemaphoreType.DMA,
        ],
        name="ncf_sc_gather",
    )
    return grab(uidx2d, iidx2d, user_table, item_table)


def _dgT(w, xT):
    return lax.dot_general(w, xT, (((0,), (0,)), ((), ())),
                           preferred_element_type=jnp.float32)


def _mlp_body(ue_ref, ie_ref, w1a, w1b, b1, w2, b2, w3, b3, wo, bo, out_ref):
    ueT = ue_ref[...].T
    ieT = ie_ref[...].T
    h = jnp.maximum(_dgT(w1a[...], ueT) + _dgT(w1b[...], ieT) + b1[...], 0.0)
    h = jnp.maximum(_dgT(w2[...], h) + b2[...], 0.0)
    h = jnp.maximum(_dgT(w3[...], h) + b3[...], 0.0)
    out_ref[...] = _dgT(wo[...], h) + bo[...]


_BM = 8192


def _tc_mlp(ue, ie, w1a, w1b, b1, w2, b2, w3, b3, wo, bo, nrows):
    full = lambda shape: pl.BlockSpec(shape, lambda i: (0, 0))
    return pl.pallas_call(
        _mlp_body,
        grid=(nrows // _BM,),
        in_specs=[
            pl.BlockSpec((_BM, _D), lambda i: (i, 0)),
            pl.BlockSpec((_BM, _D), lambda i: (i, 0)),
            full((128, 128)), full((128, 128)), full((128, 1)),
            full((128, 64)), full((64, 1)),
            full((64, 32)), full((32, 1)),
            full((32, 1)), full((1, 1)),
        ],
        out_specs=pl.BlockSpec((1, _BM), lambda i: (0, i)),
        out_shape=jax.ShapeDtypeStruct((1, nrows), jnp.float32),
        compiler_params=pltpu.CompilerParams(
            dimension_semantics=("parallel",)),
        name="ncf_tc_mlp",
    )(ue, ie, w1a, w1b, b1, w2, b2, w3, b3, wo, bo)


def kernel(user_indices, item_indices, user_table, item_table,
           W1, b1, W2, b2, W3, b3, Wo, bo):
    uidx2d = user_indices.reshape(_B // _CHUNK, _CHUNK)
    iidx2d = item_indices.reshape(_B // _CHUNK, _CHUNK)
    weights = (W1[:_D], W1[_D:], b1.reshape(-1, 1),
               W2, b2.reshape(-1, 1),
               W3, b3.reshape(-1, 1),
               Wo, bo.reshape(1, 1))
    ue3d, ie3d = _sc_gather(uidx2d, iidx2d, user_table, item_table, _B)
    ue = ue3d.reshape(_B, _D)
    ie = ie3d.reshape(_B, _D)
    out = _tc_mlp(ue, ie, *weights, _B)
    return out.reshape(_B, 1)


# speedup vs baseline: 1.5279x; 1.0334x over previous
import functools

import jax
import jax.numpy as jnp
from jax import lax
from jax.experimental import pallas as pl
from jax.experimental.pallas import tpu as pltpu
from jax.experimental.pallas import tpu_sc as plsc

_NC = 2
_NS = 16
_NW = _NC * _NS

_B = 16384
_D = 128
_CHUNK = 128
_HALF = _B // 2


def _gather_body(cpw, uidx_hbm, iidx_hbm, utab_hbm, itab_hbm, ue_out, ie_out,
                 idx_u, idx_i, buf_a, buf_b, gsem, ssem):
    wid = lax.axis_index("s") * _NC + lax.axis_index("c")
    base = wid * cpw
    pltpu.sync_copy(uidx_hbm.at[pl.ds(base, cpw)], idx_u)
    pltpu.sync_copy(iidx_hbm.at[pl.ds(base, cpw)], idx_i)
    half = cpw // 2
    gs = [pltpu.async_copy(utab_hbm.at[idx_u.at[j]], buf_a.at[j], gsem)
          for j in range(cpw)]
    for g in gs:
        g.wait()
    st_a = pltpu.async_copy(buf_a, ue_out.at[pl.ds(base, cpw)], ssem)
    gs = [pltpu.async_copy(itab_hbm.at[idx_i.at[j]], buf_b.at[j], gsem)
          for j in range(half)]
    for g in gs:
        g.wait()
    st_b = pltpu.async_copy(buf_b, ie_out.at[pl.ds(base, half)], ssem)
    st_a.wait()
    gs = [pltpu.async_copy(itab_hbm.at[idx_i.at[half + j]], buf_a.at[j], gsem)
          for j in range(half)]
    for g in gs:
        g.wait()
    st_c = pltpu.async_copy(buf_a.at[pl.ds(0, half)],
                            ie_out.at[pl.ds(base + half, half)], ssem)
    st_b.wait()
    st_c.wait()


def _sc_gather(uidx2d, iidx2d, user_table, item_table, nrows):
    cpw = nrows // _NW // _CHUNK
    mesh = plsc.VectorSubcoreMesh(core_axis_name="c", subcore_axis_name="s",
                                  num_cores=_NC, num_subcores=_NS)
    grab = pl.kernel(
        functools.partial(_gather_body, cpw),
        out_type=[
            jax.ShapeDtypeStruct((nrows // _CHUNK, _CHUNK, _D), jnp.float32),
            jax.ShapeDtypeStruct((nrows // _CHUNK, _CHUNK, _D), jnp.float32),
        ],
        mesh=mesh,
        scratch_types=[
            pltpu.VMEM((cpw, _CHUNK), jnp.int32),
            pltpu.VMEM((cpw, _CHUNK), jnp.int32),
            pltpu.VMEM((cpw, _CHUNK, _D), jnp.float32),
            pltpu.VMEM((cpw // 2, _CHUNK, _D), jnp.float32),
            pltpu.SemaphoreType.DMA,
            pltpu.SemaphoreType.DMA,
        ],
        name="ncf_sc_gather",
    )
    return grab(uidx2d, iidx2d, user_table, item_table)


def _dgT(w, xT):
    return lax.dot_general(w, xT, (((0,), (0,)), ((), ())),
                           preferred_element_type=jnp.float32)


def _mlp_body(ue_ref, ie_ref, w1a, w1b, b1, w2, b2, w3, b3, wo, bo, out_ref):
    ueT = ue_ref[...].T
    ieT = ie_ref[...].T
    h = jnp.maximum(_dgT(w1a[...], ueT) + _dgT(w1b[...], ieT) + b1[...], 0.0)
    h = jnp.maximum(_dgT(w2[...], h) + b2[...], 0.0)
    h = jnp.maximum(_dgT(w3[...], h) + b3[...], 0.0)
    out_ref[...] = _dgT(wo[...], h) + bo[...]


_BM = 8192


def _tc_mlp(ue, ie, w1a, w1b, b1, w2, b2, w3, b3, wo, bo, nrows):
    full = lambda shape: pl.BlockSpec(shape, lambda i: (0, 0))
    return pl.pallas_call(
        _mlp_body,
        grid=(nrows // _BM,),
        in_specs=[
            pl.BlockSpec((_BM, _D), lambda i: (i, 0)),
            pl.BlockSpec((_BM, _D), lambda i: (i, 0)),
            full((128, 128)), full((128, 128)), full((128, 1)),
            full((128, 64)), full((64, 1)),
            full((64, 32)), full((32, 1)),
            full((32, 1)), full((1, 1)),
        ],
        out_specs=pl.BlockSpec((1, _BM), lambda i: (0, i)),
        out_shape=jax.ShapeDtypeStruct((1, nrows), jnp.float32),
        compiler_params=pltpu.CompilerParams(
            dimension_semantics=("parallel",)),
        name="ncf_tc_mlp",
    )(ue, ie, w1a, w1b, b1, w2, b2, w3, b3, wo, bo)


def kernel(user_indices, item_indices, user_table, item_table,
           W1, b1, W2, b2, W3, b3, Wo, bo):
    uidx2d = user_indices.reshape(_B // _CHUNK, _CHUNK)
    iidx2d = item_indices.reshape(_B // _CHUNK, _CHUNK)
    weights = (W1[:_D], W1[_D:], b1.reshape(-1, 1),
               W2, b2.reshape(-1, 1),
               W3, b3.reshape(-1, 1),
               Wo, bo.reshape(1, 1))
    ue3d, ie3d = _sc_gather(uidx2d, iidx2d, user_table, item_table, _B)
    ue = ue3d.reshape(_B, _D)
    ie = ie3d.reshape(_B, _D)
    out = _tc_mlp(ue, ie, *weights, _B)
    return out.reshape(_B, 1)
